# Initial kernel scaffold; baseline (speedup 1.0000x reference)
import jax
import jax.numpy as jnp
from jax.experimental import pallas as pl


def kernel(X, c, token_table, category_table, pos_table):
    raise NotImplementedError("write your pallas kernel here")


# speedup vs baseline: 1.6270x; 1.6270x over previous
import functools

import jax
import jax.numpy as jnp
from jax import lax
from jax.experimental import pallas as pl
from jax.experimental.pallas import tpu as pltpu
from jax.experimental.pallas import tpu_sc as plsc

MAXLEN = 50
EMBED_DIM = 64
BATCH = 16384

NC = 2
NS = 16
NW = NC * NS
ROWS_PER_W = BATCH // NW
CB = 16
NCHUNK = ROWS_PER_W // CB
NV = EMBED_DIM // 16


def _sc_body(x_hbm, c_hbm, tok_hbm, cat_hbm, pos_hbm, out_hbm,
             xidx, cidx, tokbuf, catbuf, posbuf, sem):
    wid = lax.axis_index("s") * NC + lax.axis_index("c")
    rbase = wid * ROWS_PER_W

    pltpu.sync_copy(pos_hbm, posbuf)

    def chunk(ch, carry):
        base = pl.multiple_of(rbase + ch * CB, CB)
        pltpu.sync_copy(x_hbm.at[pl.ds(base, CB)], xidx)
        pltpu.sync_copy(c_hbm.at[pl.ds(base, CB)], cidx)
        cps = [pltpu.async_copy(cat_hbm.at[cidx], catbuf, sem)]
        for j in range(CB):
            cps.append(pltpu.async_copy(tok_hbm.at[xidx.at[j]],
                                        tokbuf.at[pl.ds(j * MAXLEN, MAXLEN)],
                                        sem))
        for cp in cps:
            cp.wait()

        for j in range(CB):
            catv = tuple(catbuf[j, pl.ds(v * 16, 16)] for v in range(NV))

            def lbody(l, cv, j=j):
                row = j * MAXLEN + l
                for v in range(NV):
                    sl = pl.ds(v * 16, 16)
                    tokbuf[row, sl] = tokbuf[row, sl] + posbuf[l, sl] + cv[v]
                return cv

            lax.fori_loop(0, MAXLEN, lbody, catv)

        pltpu.sync_copy(tokbuf, out_hbm.at[pl.ds(base * MAXLEN, CB * MAXLEN)])
        return carry

    lax.fori_loop(0, NCHUNK, chunk, 0)


def kernel(X, c, token_table, category_table, pos_table):
    c_flat = c.reshape(BATCH)
    mesh = plsc.VectorSubcoreMesh(core_axis_name="c", subcore_axis_name="s")
    run = pl.kernel(
        _sc_body,
        mesh=mesh,
        compiler_params=pltpu.CompilerParams(use_tc_tiling_on_sc=False),
        out_type=jax.ShapeDtypeStruct((BATCH * MAXLEN, EMBED_DIM), jnp.float32),
        scratch_types=[
            pltpu.VMEM((CB, MAXLEN), jnp.int32),
            pltpu.VMEM((CB,), jnp.int32),
            pltpu.VMEM((CB * MAXLEN, EMBED_DIM), jnp.float32),
            pltpu.VMEM((CB, EMBED_DIM), jnp.float32),
            pltpu.VMEM((MAXLEN, EMBED_DIM), jnp.float32),
            pltpu.SemaphoreType.DMA,
        ],
    )
    out = run(X, c_flat, token_table, category_table, pos_table)
    return out.reshape(BATCH, MAXLEN, EMBED_DIM)


# speedup vs baseline: 1.8546x; 1.1399x over previous
import jax
import jax.numpy as jnp
from jax import lax
from jax.experimental import pallas as pl
from jax.experimental.pallas import tpu as pltpu
from jax.experimental.pallas import tpu_sc as plsc

MAXLEN = 50
EMBED_DIM = 64
BATCH = 16384

NC = 2
NS = 16
NW = NC * NS
ROWS_PER_W = BATCH // NW
CB = 8
NCHUNK = ROWS_PER_W // CB
NBUF = 4
NV = EMBED_DIM // 16
IW = 100
IR = CB * MAXLEN // IW


def _sc_body(x_hbm, c_hbm, tok_hbm, cat_hbm, pos_hbm, out_hbm,
             xidx, cidx, tokbuf, catbuf, posbuf, gsem, outsem):
    wid = lax.axis_index("s") * NC + lax.axis_index("c")
    rbase = wid * ROWS_PER_W

    pltpu.sync_copy(pos_hbm, posbuf)

    def stage(g, b):
        base = pl.multiple_of(rbase + g * CB, CB)
        pltpu.sync_copy(x_hbm.at[pl.ds(base * MAXLEN // IW, IR)], xidx[b])
        pltpu.sync_copy(c_hbm.at[pl.ds(base, CB)], cidx[b])
        pltpu.async_copy(cat_hbm.at[cidx[b]], catbuf[b], gsem[b])
        for j in range(IR):
            pltpu.async_copy(tok_hbm.at[xidx[b].at[j]],
                             tokbuf[b].at[pl.ds(j * IW, IW)], gsem[b])

    def body(g, b, prefetch=True, outwait=True):
        base = pl.multiple_of(rbase + g * CB, CB)
        pltpu.make_async_copy(cat_hbm.at[cidx[b]], catbuf[b], gsem[b]).wait()
        for j in range(IR):
            pltpu.make_async_copy(tok_hbm.at[xidx[b].at[j]],
                                  tokbuf[b].at[pl.ds(j * IW, IW)],
                                  gsem[b]).wait()

        catv = tuple(catbuf[b][j, pl.ds(v * 16, 16)]
                     for j in range(CB) for v in range(NV))

        def lbody(l, cv):
            pos = tuple(posbuf[l, pl.ds(v * 16, 16)] for v in range(NV))
            for j in range(CB):
                row = j * MAXLEN + l
                for v in range(NV):
                    sl = pl.ds(v * 16, 16)
                    tokbuf[b][row, sl] = (tokbuf[b][row, sl] + pos[v]
                                          + cv[j * NV + v])
            return cv

        lax.fori_loop(0, MAXLEN, lbody, catv)
        pltpu.async_copy(tokbuf[b],
                         out_hbm.at[pl.ds(base * MAXLEN, CB * MAXLEN)],
                         outsem[b])
        if prefetch:
            b2 = (b + 2) % NBUF
            if outwait:
                pltpu.make_async_copy(
                    tokbuf[b2],
                    out_hbm.at[pl.ds(base * MAXLEN, CB * MAXLEN)],
                    outsem[b2]).wait()
            stage(g + 2, b2)

    stage(0, 0)
    stage(1, 1)
    body(0, 0, outwait=False)
    body(1, 1, outwait=False)
    body(2, 2)
    body(3, 3)

    def quad(k, carry):
        g = k * NBUF
        for i in range(NBUF):
            body(g + i, i)
        return carry

    lax.fori_loop(1, NCHUNK // NBUF - 1, quad, 0)
    body(NCHUNK - 4, 0)
    body(NCHUNK - 3, 1)
    body(NCHUNK - 2, 2, prefetch=False)
    body(NCHUNK - 1, 3, prefetch=False)
    for b in range(NBUF):
        g = NCHUNK - 4 + b
        base = pl.multiple_of(rbase + g * CB, CB)
        pltpu.make_async_copy(tokbuf[b],
                              out_hbm.at[pl.ds(base * MAXLEN, CB * MAXLEN)],
                              outsem[b]).wait()


def kernel(X, c, token_table, category_table, pos_table):
    x2 = X.reshape(BATCH * MAXLEN // IW, IW)
    c_flat = c.reshape(BATCH)
    mesh = plsc.VectorSubcoreMesh(core_axis_name="c", subcore_axis_name="s")
    run = pl.kernel(
        _sc_body,
        mesh=mesh,
        compiler_params=pltpu.CompilerParams(use_tc_tiling_on_sc=False),
        out_type=jax.ShapeDtypeStruct((BATCH * MAXLEN, EMBED_DIM), jnp.float32),
        scratch_types=[
            [pltpu.VMEM((IR, IW), jnp.int32) for _ in range(NBUF)],
            [pltpu.VMEM((CB,), jnp.int32) for _ in range(NBUF)],
            [pltpu.VMEM((CB * MAXLEN, EMBED_DIM), jnp.float32)
             for _ in range(NBUF)],
            [pltpu.VMEM((CB, EMBED_DIM), jnp.float32) for _ in range(NBUF)],
            pltpu.VMEM((MAXLEN, EMBED_DIM), jnp.float32),
            [pltpu.SemaphoreType.DMA for _ in range(NBUF)],
            [pltpu.SemaphoreType.DMA for _ in range(NBUF)],
        ],
    )
    out = run(x2, c_flat, token_table, category_table, pos_table)
    return out.reshape(BATCH, MAXLEN, EMBED_DIM)
